# Initial kernel scaffold; baseline (speedup 1.0000x reference)
import jax
import jax.numpy as jnp
from jax.experimental import pallas as pl


def kernel(input_coords, input_feats):
    raise NotImplementedError("write your pallas kernel here")


# speedup vs baseline: 4.3818x; 4.3818x over previous
import functools

import jax
import jax.numpy as jnp
from jax.experimental import pallas as pl
from jax.experimental.pallas import tpu as pltpu

_M = 1024
_K = 16
_MC = 256
_BIG = 3.0e38


def _body(coords_ref, nodes_ref, ft_ref, out_ref, s_ref):
    x = coords_ref[0]
    nd = nodes_ref[0]
    ft = ft_ref[0]

    s = None
    for c in range(3):
        d = x[c : c + 1, :] - nd[:, c : c + 1]
        s = d * d if s is None else s + d * d
    s_ref[...] = s

    def it(_, out):
        s = s_ref[...]
        m = jnp.min(s, axis=1, keepdims=True)
        oh = s <= m
        g = jax.lax.dot_general(
            oh.astype(jnp.float32), ft,
            (((1,), (0,)), ((), ())),
            preferred_element_type=jnp.float32,
            precision=jax.lax.Precision.HIGHEST,
        )
        s_ref[...] = jnp.where(oh, _BIG, s)
        return jnp.maximum(out, g)

    C = ft.shape[1]
    out = jax.lax.fori_loop(
        0, _K, it, jnp.full((nd.shape[0], C), -_BIG, jnp.float32))
    out_ref[0] = out


@jax.jit
def kernel(input_coords, input_feats):
    B, _, N = input_coords.shape
    C = input_feats.shape[1]
    coords_p = jnp.pad(input_coords, ((0, 0), (0, 5), (0, 0)))
    nodes_t = jnp.transpose(coords_p[:, :, :_M], (0, 2, 1))
    ft = jnp.transpose(input_feats, (0, 2, 1))

    grid = (B, _M // _MC)
    agg_t = pl.pallas_call(
        _body,
        grid=grid,
        in_specs=[
            pl.BlockSpec((1, 8, N), lambda b, m: (b, 0, 0)),
            pl.BlockSpec((1, _MC, 8), lambda b, m: (b, m, 0)),
            pl.BlockSpec((1, N, C), lambda b, m: (b, 0, 0)),
        ],
        out_specs=pl.BlockSpec((1, _MC, C), lambda b, m: (b, m, 0)),
        out_shape=jax.ShapeDtypeStruct((B, _M, C), jnp.float32),
        scratch_shapes=[pltpu.VMEM((_MC, N), jnp.float32)],
    )(coords_p, nodes_t, ft)

    agg = jnp.transpose(agg_t, (0, 2, 1))
    pool_coords = input_coords[:, :, :_M]
    pool_feats = jnp.concatenate((input_feats[:, :, :_M], agg), axis=1)
    return (pool_coords, pool_coords, pool_feats)


# speedup vs baseline: 9.5027x; 2.1687x over previous
import functools

import jax
import jax.numpy as jnp
from jax.experimental import pallas as pl
from jax.experimental.pallas import tpu as pltpu

_M = 1024
_K = 16
_MC = 256
_BIG = 3.0e38


def _dot(a, b):
    return jax.lax.dot_general(
        a, b, (((1,), (0,)), ((), ())),
        preferred_element_type=jnp.float32)


def _body(coords_ref, nodes_ref, fhi_ref, flo_ref, out_ref, s_ref):
    x = coords_ref[0]
    nd = nodes_ref[0]
    fhi = fhi_ref[0]
    flo = flo_ref[0]

    s = None
    for c in range(3):
        d = x[c : c + 1, :] - nd[:, c : c + 1]
        s = d * d if s is None else s + d * d
    s_ref[...] = s

    def it(_, out):
        s = s_ref[...]
        m = jnp.min(s, axis=1, keepdims=True)
        oh = s <= m
        ohb = oh.astype(jnp.bfloat16)
        g = _dot(ohb, fhi) + _dot(ohb, flo)
        s_ref[...] = jnp.where(oh, _BIG, s)
        return jnp.maximum(out, g)

    C = fhi.shape[1]
    out = jax.lax.fori_loop(
        0, _K, it, jnp.full((nd.shape[0], C), -_BIG, jnp.float32))
    out_ref[0] = out


@jax.jit
def kernel(input_coords, input_feats):
    B, _, N = input_coords.shape
    C = input_feats.shape[1]
    coords_p = jnp.pad(input_coords, ((0, 0), (0, 5), (0, 0)))
    nodes_t = jnp.transpose(coords_p[:, :, :_M], (0, 2, 1))
    ft = jnp.transpose(input_feats, (0, 2, 1))
    ft_hi = ft.astype(jnp.bfloat16)
    ft_lo = (ft - ft_hi.astype(jnp.float32)).astype(jnp.bfloat16)

    grid = (B, _M // _MC)
    agg_t = pl.pallas_call(
        _body,
        grid=grid,
        in_specs=[
            pl.BlockSpec((1, 8, N), lambda b, m: (b, 0, 0)),
            pl.BlockSpec((1, _MC, 8), lambda b, m: (b, m, 0)),
            pl.BlockSpec((1, N, C), lambda b, m: (b, 0, 0)),
            pl.BlockSpec((1, N, C), lambda b, m: (b, 0, 0)),
        ],
        out_specs=pl.BlockSpec((1, _MC, C), lambda b, m: (b, m, 0)),
        out_shape=jax.ShapeDtypeStruct((B, _M, C), jnp.float32),
        scratch_shapes=[pltpu.VMEM((_MC, N), jnp.float32)],
    )(coords_p, nodes_t, ft_hi, ft_lo)

    agg = jnp.transpose(agg_t, (0, 2, 1))
    pool_coords = input_coords[:, :, :_M]
    pool_feats = jnp.concatenate((input_feats[:, :, :_M], agg), axis=1)
    return (pool_coords, pool_coords, pool_feats)


# speedup vs baseline: 13.3990x; 1.4100x over previous
import functools

import jax
import jax.numpy as jnp
from jax.experimental import pallas as pl
from jax.experimental.pallas import tpu as pltpu

_M = 1024
_K = 16
_MC = 256
_BIG = 3.0e38


def _dot(a, b):
    return jax.lax.dot_general(
        a, b, (((1,), (0,)), ((), ())),
        preferred_element_type=jnp.float32)


def _body(coords_ref, nodes_ref, fhi_ref, flo_ref, out_ref, s_ref):
    x = coords_ref[0]
    nd = nodes_ref[0]
    fhi = fhi_ref[0]
    flo = flo_ref[0]

    s = None
    for c in range(3):
        d = x[c : c + 1, :] - nd[:, c : c + 1]
        s = d * d if s is None else s + d * d
    s_ref[...] = s

    def it(_, out):
        s = s_ref[...]
        m = jnp.min(s, axis=1, keepdims=True)
        oh = s <= m
        ohb = oh.astype(jnp.bfloat16)
        g = _dot(ohb, fhi)
        s_ref[...] = jnp.where(oh, _BIG, s)
        return jnp.maximum(out, g)

    C = fhi.shape[1]
    out = jax.lax.fori_loop(
        0, _K, it, jnp.full((nd.shape[0], C), -_BIG, jnp.float32))
    out_ref[0] = out


@jax.jit
def kernel(input_coords, input_feats):
    B, _, N = input_coords.shape
    C = input_feats.shape[1]
    coords_p = jnp.pad(input_coords, ((0, 0), (0, 5), (0, 0)))
    nodes_t = jnp.transpose(coords_p[:, :, :_M], (0, 2, 1))
    ft = jnp.transpose(input_feats, (0, 2, 1))
    ft_hi = ft.astype(jnp.bfloat16)
    ft_lo = (ft - ft_hi.astype(jnp.float32)).astype(jnp.bfloat16)

    grid = (B, _M // _MC)
    agg_t = pl.pallas_call(
        _body,
        grid=grid,
        in_specs=[
            pl.BlockSpec((1, 8, N), lambda b, m: (b, 0, 0)),
            pl.BlockSpec((1, _MC, 8), lambda b, m: (b, m, 0)),
            pl.BlockSpec((1, N, C), lambda b, m: (b, 0, 0)),
            pl.BlockSpec((1, N, C), lambda b, m: (b, 0, 0)),
        ],
        out_specs=pl.BlockSpec((1, _MC, C), lambda b, m: (b, m, 0)),
        out_shape=jax.ShapeDtypeStruct((B, _M, C), jnp.float32),
        scratch_shapes=[pltpu.VMEM((_MC, N), jnp.float32)],
    )(coords_p, nodes_t, ft_hi, ft_lo)

    agg = jnp.transpose(agg_t, (0, 2, 1))
    pool_coords = input_coords[:, :, :_M]
    pool_feats = jnp.concatenate((input_feats[:, :, :_M], agg), axis=1)
    return (pool_coords, pool_coords, pool_feats)
